# baseline (speedup 1.0000x reference)
import functools

import jax
import jax.numpy as jnp
from jax import lax
from jax.experimental import pallas as pl
from jax.experimental.pallas import tpu as pltpu
from jax.experimental.pallas import tpu_sc as plsc

_NUM_CORES = 2
_NUM_SUBCORES = 16
_NW = _NUM_CORES * _NUM_SUBCORES
_CHUNK = 64


def _sc_bcast_body(table_hbm, out_hbm, buf):
    batch = out_hbm.shape[0]
    seq = table_hbm.shape[0]
    rows_per_w = seq // _NW
    wid = lax.axis_index("s") * _NUM_CORES + lax.axis_index("c")
    base = wid * rows_per_w
    for c in range(rows_per_w // _CHUNK):
        r0 = base + c * _CHUNK
        pltpu.sync_copy(table_hbm.at[pl.ds(r0, _CHUNK)], buf)
        for b in range(batch):
            pltpu.sync_copy(buf, out_hbm.at[b, pl.ds(r0, _CHUNK)])


def kernel(x, table):
    batch = x.shape[0]
    seq, dim = table.shape
    mesh = plsc.VectorSubcoreMesh(
        core_axis_name="c", subcore_axis_name="s",
        num_cores=_NUM_CORES, num_subcores=_NUM_SUBCORES)
    sc_call = pl.kernel(
        _sc_bcast_body, mesh=mesh,
        out_type=jax.ShapeDtypeStruct((batch, seq, dim), table.dtype),
        scratch_types=[pltpu.VMEM((_CHUNK, dim), table.dtype)],
    )
    return sc_call(table)


# speedup vs baseline: 1.4366x; 1.4366x over previous
import functools

import jax
import jax.numpy as jnp
from jax import lax
from jax.experimental import pallas as pl
from jax.experimental.pallas import tpu as pltpu
from jax.experimental.pallas import tpu_sc as plsc

_NUM_CORES = 2
_NUM_SUBCORES = 16
_NW = _NUM_CORES * _NUM_SUBCORES
_CHUNK = 64


def _sc_bcast_body(table_hbm, out_hbm, buf):
    batch = out_hbm.shape[0]
    seq = table_hbm.shape[0]
    rows_per_w = seq // _NW
    wid = lax.axis_index("s") * _NUM_CORES + lax.axis_index("c")
    base = wid * rows_per_w
    for c in range(rows_per_w // _CHUNK):
        r0 = base + c * _CHUNK
        pltpu.sync_copy(table_hbm.at[pl.ds(r0, _CHUNK)], buf)
        for b in range(batch):
            pltpu.sync_copy(buf, out_hbm.at[b, pl.ds(r0, _CHUNK)])


def _tc_bcast_body(t_ref, o_ref):
    o_ref[...] = jnp.broadcast_to(t_ref[...][None], o_ref.shape)


def kernel(x, table):
    batch = x.shape[0]
    seq, dim = table.shape
    bs = 1024
    out = pl.pallas_call(
        _tc_bcast_body,
        grid=(seq // bs,),
        in_specs=[pl.BlockSpec((bs, dim), lambda s: (s, 0))],
        out_specs=pl.BlockSpec((batch, bs, dim), lambda s: (0, s, 0)),
        out_shape=jax.ShapeDtypeStruct((batch, seq, dim), table.dtype),
    )(table)
    return out
